# baseline (speedup 1.0000x reference)
import jax
import jax.numpy as jnp
from jax import lax
from jax.experimental import pallas as pl
from jax.experimental.pallas import tpu as pltpu
from jax.experimental.pallas import tpu_sc as plsc

_T = 1000
_B = 512
_F = 4 * 64 * 64
_LANES = 16
_D = 128
_NC, _NS = 2, 16
_NW = _NC * _NS
_BPW = _B // _NW


def _sc_gather(ac, om, t):
    mesh = plsc.VectorSubcoreMesh(core_axis_name="c", subcore_axis_name="s")

    @pl.kernel(
        mesh=mesh,
        out_type=jax.ShapeDtypeStruct((8, _B), jnp.float32),
        scratch_types=[
            pltpu.VMEM((_BPW,), jnp.int32),
            pltpu.VMEM((_BPW,), jnp.float32),
            pltpu.VMEM((_BPW,), jnp.float32),
            pltpu.SemaphoreType.DMA,
            pltpu.SemaphoreType.DMA,
        ],
        compiler_params=pltpu.CompilerParams(
            use_tc_tiling_on_sc=False, skip_device_barrier=True
        ),
    )
    def k(ac_hbm, om_hbm, t_hbm, st_hbm, idx_v, a_v, b_v, sem_a, sem_b):
        wid = lax.axis_index("s") * _NC + lax.axis_index("c")
        base = wid * _BPW
        pltpu.sync_copy(t_hbm.at[pl.ds(base, _BPW)], idx_v)
        ca = pltpu.async_copy(ac_hbm.at[idx_v], a_v, sem_a)
        cb = pltpu.async_copy(om_hbm.at[idx_v], b_v, sem_b)
        ca.wait()
        cb.wait()
        pltpu.sync_copy(a_v, st_hbm.at[0, pl.ds(base, _BPW)])
        pltpu.sync_copy(b_v, st_hbm.at[1, pl.ds(base, _BPW)])

    return k(ac, om, t)


_ROWS = 32
_NCHUNK = _B // _ROWS
_NBUF = 4


def _tc_body(s_ref, x_ref, n_ref, o_ref, xbuf, nbuf, obuf, xsem, nsem, osem):
    def start_in(i, slot):
        rows = pl.ds(i * _ROWS, _ROWS)
        pltpu.make_async_copy(x_ref.at[rows], xbuf.at[slot], xsem.at[slot]).start()
        pltpu.make_async_copy(n_ref.at[rows], nbuf.at[slot], nsem.at[slot]).start()

    for k in range(_NBUF):
        start_in(k, k)

    for i in range(_NCHUNK):
        slot = i % _NBUF
        rows = pl.ds(i * _ROWS, _ROWS)
        pltpu.make_async_copy(x_ref.at[rows], xbuf.at[slot], xsem.at[slot]).wait()
        pltpu.make_async_copy(n_ref.at[rows], nbuf.at[slot], nsem.at[slot]).wait()
        if i >= _NBUF:
            pltpu.make_async_copy(obuf.at[slot], o_ref.at[rows], osem.at[slot]).wait()
        a = s_ref[i * _ROWS:(i + 1) * _ROWS, 0:1]
        b = s_ref[i * _ROWS:(i + 1) * _ROWS, 1:2]
        obuf[slot] = a * xbuf[slot] + b * nbuf[slot]
        pltpu.make_async_copy(obuf.at[slot], o_ref.at[rows], osem.at[slot]).start()
        nxt = i + _NBUF
        if nxt < _NCHUNK:
            start_in(nxt, slot)

    for i in range(_NCHUNK - _NBUF, _NCHUNK):
        slot = i % _NBUF
        rows = pl.ds(i * _ROWS, _ROWS)
        pltpu.make_async_copy(obuf.at[slot], o_ref.at[rows], osem.at[slot]).wait()


def _tc_scale_add(st, x, n):
    rows = 2048
    grid = (_F // rows,)

    def body(st_ref, x_ref, n_ref, o_ref):
        a = st_ref[0:1, :]
        b = st_ref[1:2, :]
        o_ref[...] = a * x_ref[...] + b * n_ref[...]

    return pl.pallas_call(
        body,
        grid=grid,
        in_specs=[
            pl.BlockSpec((8, _B), lambda i: (0, 0)),
            pl.BlockSpec((rows, _B), lambda i: (i, 0)),
            pl.BlockSpec((rows, _B), lambda i: (i, 0)),
        ],
        out_specs=pl.BlockSpec((rows, _B), lambda i: (i, 0)),
        out_shape=jax.ShapeDtypeStruct((_F, _B), jnp.float32),
        compiler_params=pltpu.CompilerParams(
            dimension_semantics=("parallel",), skip_device_barrier=True,
        ),
    )(st, x, n)


def kernel(x0, t, noise, sqrt_alphas_cumprod, sqrt_one_minus_alphas_cumprod):
    st = _sc_gather(
        sqrt_alphas_cumprod, sqrt_one_minus_alphas_cumprod, t.astype(jnp.int32)
    )
    x = jnp.transpose(x0, (1, 2, 3, 0)).reshape(_F, _B)
    n = jnp.transpose(noise, (1, 2, 3, 0)).reshape(_F, _B)
    out = _tc_scale_add(st, x, n)
    return jnp.transpose(out.reshape(4, 64, 64, _B), (3, 0, 1, 2))


# speedup vs baseline: 1.0038x; 1.0038x over previous
import jax
import jax.numpy as jnp
from jax import lax
from jax.experimental import pallas as pl
from jax.experimental.pallas import tpu as pltpu
from jax.experimental.pallas import tpu_sc as plsc

_B = 512
_F = 4 * 64 * 64
_NC, _NS = 2, 16
_NW = _NC * _NS
_BPW = _B // _NW


def _sc_gather(ac, om, t):
    mesh = plsc.VectorSubcoreMesh(core_axis_name="c", subcore_axis_name="s")

    @pl.kernel(
        mesh=mesh,
        out_type=jax.ShapeDtypeStruct((8, _B), jnp.float32),
        scratch_types=[
            pltpu.VMEM((_BPW,), jnp.int32),
            pltpu.VMEM((_BPW,), jnp.float32),
            pltpu.VMEM((_BPW,), jnp.float32),
            pltpu.SemaphoreType.DMA,
            pltpu.SemaphoreType.DMA,
        ],
        compiler_params=pltpu.CompilerParams(use_tc_tiling_on_sc=False),
    )
    def k(ac_hbm, om_hbm, t_hbm, st_hbm, idx_v, a_v, b_v, sem_a, sem_b):
        wid = lax.axis_index("s") * _NC + lax.axis_index("c")
        base = wid * _BPW
        pltpu.sync_copy(t_hbm.at[pl.ds(base, _BPW)], idx_v)
        ca = pltpu.async_copy(ac_hbm.at[idx_v], a_v, sem_a)
        cb = pltpu.async_copy(om_hbm.at[idx_v], b_v, sem_b)
        ca.wait()
        cb.wait()
        pltpu.sync_copy(a_v, st_hbm.at[0, pl.ds(base, _BPW)])
        pltpu.sync_copy(b_v, st_hbm.at[1, pl.ds(base, _BPW)])

    return k(ac, om, t)


def _tc_scale_add(st, x, n):
    rows = 2048
    grid = (_F // rows,)

    def body(st_ref, x_ref, n_ref, o_ref):
        a = st_ref[0:1, :]
        b = st_ref[1:2, :]
        o_ref[...] = a * x_ref[...] + b * n_ref[...]

    return pl.pallas_call(
        body,
        grid=grid,
        in_specs=[
            pl.BlockSpec((8, _B), lambda i: (0, 0)),
            pl.BlockSpec((rows, _B), lambda i: (i, 0)),
            pl.BlockSpec((rows, _B), lambda i: (i, 0)),
        ],
        out_specs=pl.BlockSpec((rows, _B), lambda i: (i, 0)),
        out_shape=jax.ShapeDtypeStruct((_F, _B), jnp.float32),
        compiler_params=pltpu.CompilerParams(
            dimension_semantics=("parallel",),
        ),
    )(st, x, n)


def kernel(x0, t, noise, sqrt_alphas_cumprod, sqrt_one_minus_alphas_cumprod):
    st = _sc_gather(
        sqrt_alphas_cumprod, sqrt_one_minus_alphas_cumprod, t.astype(jnp.int32)
    )
    x = jnp.transpose(x0, (1, 2, 3, 0)).reshape(_F, _B)
    n = jnp.transpose(noise, (1, 2, 3, 0)).reshape(_F, _B)
    out = _tc_scale_add(st, x, n)
    return jnp.transpose(out.reshape(4, 64, 64, _B), (3, 0, 1, 2))
